# baseline (speedup 1.0000x reference)
import functools

import jax
import jax.numpy as jnp
from jax import lax
from jax.experimental import pallas as pl
from jax.experimental.pallas import tpu as pltpu
from jax.experimental.pallas import tpu_sc as plsc

VOCAB = 1000002
EMB = 32
BATCH = 16384
SEQ = 200

NUM_CORES = 2
NUM_SUBCORES = 16
NW = NUM_CORES * NUM_SUBCORES
SENT_PER_W = BATCH // NW
CHUNK = 128
NCHUNK = SENT_PER_W // CHUNK
_H0 = 96
_H1 = SEQ - _H0

_TW_BLOCK = 131072
_TW_GRID = (VOCAB + _TW_BLOCK - 1) // _TW_BLOCK
VPAD = _TW_GRID * _TW_BLOCK
_STAGE = VPAD // NUM_SUBCORES


def _tw_body(tabt_ref, w_ref, o_ref):
    o_ref[...] = jnp.sum(tabt_ref[...] * w_ref[...], axis=0)


def _table_times_w(table_t, w):
    return pl.pallas_call(
        _tw_body,
        grid=(_TW_GRID,),
        in_specs=[
            pl.BlockSpec((EMB, _TW_BLOCK), lambda i: (0, i)),
            pl.BlockSpec((EMB, 1), lambda i: (0, 0)),
        ],
        out_specs=pl.BlockSpec((_TW_BLOCK,), lambda i: (i,)),
        out_shape=jax.ShapeDtypeStruct((VPAD,), jnp.float32),
    )(table_t, w.reshape(EMB, 1))


_mesh = plsc.VectorSubcoreMesh(core_axis_name="c", subcore_axis_name="s")


@functools.partial(
    pl.kernel,
    out_type=jax.ShapeDtypeStruct((BATCH,), jnp.float32),
    mesh=_mesh,
    compiler_params=pltpu.CompilerParams(needs_layout_passes=False),
    scratch_types=[
        pltpu.VMEM_SHARED((VPAD,), jnp.float32),
        pltpu.VMEM((2, _H1, CHUNK), jnp.int32),
        pltpu.VMEM((2, _H1 * CHUNK), jnp.float32),
        pltpu.VMEM((CHUNK,), jnp.int32),
        pltpu.VMEM((CHUNK,), jnp.float32),
        pltpu.VMEM((16,), jnp.float32),
        pltpu.SemaphoreType.DMA,
        pltpu.SemaphoreType.DMA,
    ],
)
def _sc_pool(tw_hbm, datat_hbm, len_hbm, b_hbm, out_hbm,
             tw_sp, idx_v, vals_v, len_v, out_v, b_v, sem0, sem1):
    cid = lax.axis_index("c")
    sid = lax.axis_index("s")
    wid = sid * NUM_CORES + cid
    sems = (sem0, sem1)
    col0 = wid * SENT_PER_W

    units = [(ch, h) for ch in range(NCHUNK) for h in range(2)]

    def _load_idx(i):
        buf = i % 2
        ch, h = units[i]
        r0, nr = (0, _H0) if h == 0 else (_H0, _H1)
        pltpu.sync_copy(
            datat_hbm.at[pl.ds(r0, nr), pl.ds(col0 + ch * CHUNK, CHUNK)],
            idx_v.at[buf, pl.ds(0, nr)],
        )

    def _fire(i):
        buf = i % 2
        nr = _H0 if units[i][1] == 0 else _H1

        def body(r, carry):
            pltpu.async_copy(
                tw_sp.at[idx_v.at[buf, r]],
                vals_v.at[buf, pl.ds(pl.multiple_of(r * CHUNK, CHUNK), CHUNK)],
                sems[buf],
            )
            return carry

        lax.fori_loop(0, nr, body, 0)

    off = sid * _STAGE
    _load_idx(0)
    pltpu.sync_copy(b_hbm, b_v)
    pltpu.sync_copy(tw_hbm.at[pl.ds(off, _STAGE)], tw_sp.at[pl.ds(off, _STAGE)])
    plsc.subcore_barrier()

    bvec = b_v[...]
    zero = jnp.zeros((16,), jnp.float32)

    _fire(0)
    accs = (zero,) * 8

    for i in range(len(units)):
        buf = i % 2
        ch, h = units[i]
        nr = _H0 if h == 0 else _H1

        if i + 1 < len(units):
            _load_idx(i + 1)
            _fire(i + 1)

        pltpu.make_async_copy(
            tw_hbm.at[pl.ds(0, nr * CHUNK)],
            vals_v.at[buf, pl.ds(0, nr * CHUNK)],
            sems[buf],
        ).wait()

        def _srow(s, a):
            base = s * (2 * CHUNK)
            a = tuple(
                a[u] + vals_v[buf, pl.ds(base + 16 * u, 16)] for u in range(8)
            )
            return tuple(
                a[u] + vals_v[buf, pl.ds(base + CHUNK + 16 * u, 16)]
                for u in range(8)
            )

        accs = lax.fori_loop(0, nr // 2, _srow, accs)

        if h == 1:
            col = col0 + ch * CHUNK
            pltpu.sync_copy(len_hbm.at[pl.ds(col, CHUNK)], len_v)
            for u in range(8):
                lenf = len_v[pl.ds(u * 16, 16)].astype(jnp.float32)
                x = accs[u] / lenf + bvec
                out_v[pl.ds(u * 16, 16)] = 1.0 / (1.0 + jnp.exp(-x))
            pltpu.sync_copy(out_v, out_hbm.at[pl.ds(col, CHUNK)])
            accs = (zero,) * 8


def kernel(data, length, table, w, b):
    tw = _table_times_w(table.T, w)
    b16 = jnp.broadcast_to(b.astype(jnp.float32), (16,))
    return _sc_pool(tw, data.T, length.astype(jnp.int32), b16)


# speedup vs baseline: 1.0073x; 1.0073x over previous
import functools

import jax
import jax.numpy as jnp
from jax import lax
from jax.experimental import pallas as pl
from jax.experimental.pallas import tpu as pltpu
from jax.experimental.pallas import tpu_sc as plsc

VOCAB = 1000002
EMB = 32
BATCH = 16384
SEQ = 200

NUM_CORES = 2
NUM_SUBCORES = 16
NW = NUM_CORES * NUM_SUBCORES
SENT_PER_W = BATCH // NW
CHUNK = 128
NCHUNK = SENT_PER_W // CHUNK
_H0 = 96
_H1 = SEQ - _H0

_TW_BLOCK = 131072
_TW_GRID = (VOCAB + _TW_BLOCK - 1) // _TW_BLOCK
VPAD = _TW_GRID * _TW_BLOCK
_STAGE = VPAD // NUM_SUBCORES


def _tw_body(tabt_ref, w_ref, o_ref):
    o_ref[...] = jnp.sum(tabt_ref[...] * w_ref[...], axis=0)


def _table_times_w(table_t, w):
    return pl.pallas_call(
        _tw_body,
        grid=(_TW_GRID,),
        in_specs=[
            pl.BlockSpec((EMB, _TW_BLOCK), lambda i: (0, i)),
            pl.BlockSpec((EMB, 1), lambda i: (0, 0)),
        ],
        out_specs=pl.BlockSpec((_TW_BLOCK,), lambda i: (i,)),
        out_shape=jax.ShapeDtypeStruct((VPAD,), jnp.float32),
    )(table_t, w.reshape(EMB, 1))


_mesh = plsc.VectorSubcoreMesh(core_axis_name="c", subcore_axis_name="s")


@functools.partial(
    pl.kernel,
    out_type=jax.ShapeDtypeStruct((BATCH,), jnp.float32),
    mesh=_mesh,
    compiler_params=pltpu.CompilerParams(needs_layout_passes=False),
    scratch_types=[
        pltpu.VMEM_SHARED((VPAD,), jnp.float32),
        pltpu.VMEM((2, _H1, CHUNK), jnp.int32),
        pltpu.VMEM((2, _H1 * CHUNK), jnp.float32),
        pltpu.VMEM((CHUNK,), jnp.int32),
        pltpu.VMEM((CHUNK,), jnp.float32),
        pltpu.VMEM((16,), jnp.float32),
        pltpu.SemaphoreType.DMA,
        pltpu.SemaphoreType.DMA,
    ],
)
def _sc_pool(tw_hbm, datat_hbm, len_hbm, b_hbm, out_hbm,
             tw_sp, idx_v, vals_v, len_v, out_v, b_v, sem0, sem1):
    cid = lax.axis_index("c")
    sid = lax.axis_index("s")
    wid = sid * NUM_CORES + cid
    sems = (sem0, sem1)
    col0 = wid * SENT_PER_W

    units = [(ch, h) for ch in range(NCHUNK) for h in range(2)]

    def _load_idx(i):
        buf = i % 2
        ch, h = units[i]
        r0, nr = (0, _H0) if h == 0 else (_H0, _H1)
        pltpu.sync_copy(
            datat_hbm.at[pl.ds(r0, nr), pl.ds(col0 + ch * CHUNK, CHUNK)],
            idx_v.at[buf, pl.ds(0, nr)],
        )

    def _fire(i):
        buf = i % 2
        nr = _H0 if units[i][1] == 0 else _H1

        def body(r, carry):
            pltpu.async_copy(
                tw_sp.at[idx_v.at[buf, r]],
                vals_v.at[buf, pl.ds(pl.multiple_of(r * CHUNK, CHUNK), CHUNK)],
                sems[buf],
            )
            return carry

        lax.fori_loop(0, nr, body, 0)

    off = sid * _STAGE
    _load_idx(0)
    pltpu.sync_copy(b_hbm, b_v)
    pltpu.sync_copy(tw_hbm.at[pl.ds(off, _STAGE)], tw_sp.at[pl.ds(off, _STAGE)])
    plsc.subcore_barrier()

    bvec = b_v[...]
    zero = jnp.zeros((16,), jnp.float32)

    _fire(0)
    accs = (zero,) * 8

    for i in range(len(units)):
        buf = i % 2
        ch, h = units[i]
        nr = _H0 if h == 0 else _H1

        if i + 1 < len(units):
            _load_idx(i + 1)
            _fire(i + 1)

        pltpu.make_async_copy(
            tw_hbm.at[pl.ds(0, nr * CHUNK)],
            vals_v.at[buf, pl.ds(0, nr * CHUNK)],
            sems[buf],
        ).wait()

        def _srow(s, a):
            base = s * CHUNK
            return tuple(
                a[u] + vals_v[buf, pl.ds(base + 16 * u, 16)] for u in range(8)
            )

        accs = lax.fori_loop(0, nr, _srow, accs)

        if h == 1:
            col = col0 + ch * CHUNK
            pltpu.sync_copy(len_hbm.at[pl.ds(col, CHUNK)], len_v)
            for u in range(8):
                lenf = len_v[pl.ds(u * 16, 16)].astype(jnp.float32)
                x = accs[u] / lenf + bvec
                out_v[pl.ds(u * 16, 16)] = 1.0 / (1.0 + jnp.exp(-x))
            pltpu.sync_copy(out_v, out_hbm.at[pl.ds(col, CHUNK)])
            accs = (zero,) * 8


def kernel(data, length, table, w, b):
    tw = _table_times_w(table.T, w)
    b16 = jnp.broadcast_to(b.astype(jnp.float32), (16,))
    return _sc_pool(tw, data.T, length.astype(jnp.int32), b16)
